# Initial kernel scaffold; baseline (speedup 1.0000x reference)
import jax
import jax.numpy as jnp
from jax.experimental import pallas as pl


def kernel(h, x, edges, edge_attr, params):
    raise NotImplementedError("write your pallas kernel here")


# speedup vs baseline: 2.0384x; 2.0384x over previous
import functools

import jax
import jax.numpy as jnp
from jax import lax
from jax.experimental import pallas as pl
from jax.experimental.pallas import tpu as pltpu
from jax.experimental.pallas import tpu_sc as plsc

N = 10000
E = 320000
D = 128
DE = 16
HID = 128
XP = 16

NC, NS = 2, 16
NW = NC * NS
EPW = E // NW
CHUNK = 80
NCHUNK = EPW // CHUNK
NSTAGE = N // CHUNK
GCH = 40
GNCHUNK = EPW // GCH
GNSTAGE = N // GCH

BT = 2000
BN = 2000


def _sc_mesh():
    return plsc.VectorSubcoreMesh(core_axis_name="c", subcore_axis_name="s",
                                  num_cores=NC, num_subcores=NS)


def _gather_body(hA, hB, x128, rowi, coli, gA_o, gB_o, xd_o,
                 idx_r, idx_c, bufA, bufB, bufxr, bufxc, bufd, x_sp, sem):
    c = lax.axis_index("c")
    s = lax.axis_index("s")
    wid = s * NC + c
    base = wid * EPW

    def stage(jj, carry):
        j = s + jj * NS

        @pl.when(j < GNSTAGE)
        def _():
            r0 = j * GCH
            pltpu.sync_copy(x128.at[pl.ds(r0, GCH)], bufxr)
            pltpu.sync_copy(bufxr, x_sp.at[pl.ds(r0, GCH)])

        return carry

    lax.fori_loop(0, (GNSTAGE + NS - 1) // NS, stage, 0)
    plsc.subcore_barrier()

    def step(k, carry):
        off = base + k * GCH
        pltpu.sync_copy(rowi.at[pl.ds(off, GCH)], idx_r)
        pltpu.sync_copy(coli.at[pl.ds(off, GCH)], idx_c)
        pltpu.async_copy(hA.at[idx_r], bufA, sem).wait()
        pltpu.async_copy(hB.at[idx_c], bufB, sem).wait()
        pltpu.async_copy(x_sp.at[idx_r], bufxr, sem).wait()
        pltpu.async_copy(x_sp.at[idx_c], bufxc, sem).wait()

        def diff(i, cc):
            bufd[i, pl.ds(0, XP)] = (bufxr[i, pl.ds(0, XP)]
                                     - bufxc[i, pl.ds(0, XP)])
            return cc

        lax.fori_loop(0, GCH, diff, 0)
        pltpu.sync_copy(bufA, gA_o.at[pl.ds(off, GCH)])
        pltpu.sync_copy(bufB, gB_o.at[pl.ds(off, GCH)])
        pltpu.sync_copy(bufd, xd_o.at[pl.ds(off, GCH)])
        return carry

    lax.fori_loop(0, GNCHUNK, step, 0)


@functools.cache
def _sc_gather_kernel():
    return pl.kernel(
        _gather_body,
        out_type=(
            jax.ShapeDtypeStruct((E, HID), jnp.float32),
            jax.ShapeDtypeStruct((E, HID), jnp.float32),
            jax.ShapeDtypeStruct((E, XP), jnp.float32),
        ),
        mesh=_sc_mesh(),
        scratch_types=[
            pltpu.VMEM((GCH,), jnp.int32),
            pltpu.VMEM((GCH,), jnp.int32),
            pltpu.VMEM((GCH, HID), jnp.float32),
            pltpu.VMEM((GCH, HID), jnp.float32),
            pltpu.VMEM((GCH, HID), jnp.float32),
            pltpu.VMEM((GCH, HID), jnp.float32),
            pltpu.VMEM((GCH, XP), jnp.float32),
            pltpu.VMEM_SHARED((N, HID), jnp.float32),
            pltpu.SemaphoreType.DMA,
        ],
    )


def _scatter_h_body(m, rowi, aggh_o, idx, bufM, agg_h, sem):
    c = lax.axis_index("c")
    s = lax.axis_index("s")
    wid = s * NC + c
    base = wid * EPW

    def zrow(i, carry):
        for jc in range(HID // 16):
            bufM[i, pl.ds(jc * 16, 16)] = jnp.zeros((16,), jnp.float32)
        return carry

    lax.fori_loop(0, CHUNK, zrow, 0)

    def zstage(jj, carry):
        j = s + jj * NS

        @pl.when(j < NSTAGE)
        def _():
            pltpu.sync_copy(bufM, agg_h.at[pl.ds(j * CHUNK, CHUNK)])

        return carry

    lax.fori_loop(0, (NSTAGE + NS - 1) // NS, zstage, 0)
    plsc.subcore_barrier()

    def step(k, carry):
        off = base + k * CHUNK
        pltpu.sync_copy(rowi.at[pl.ds(off, CHUNK)], idx)
        pltpu.sync_copy(m.at[pl.ds(off, CHUNK)], bufM)
        pltpu.sync_copy(bufM, agg_h.at[idx], add=True)
        return carry

    lax.fori_loop(0, NCHUNK, step, 0)
    plsc.subcore_barrier()

    def fstage(jj, carry):
        j = s + jj * NS

        @pl.when(j < NSTAGE)
        def _():
            pltpu.sync_copy(agg_h.at[pl.ds(j * CHUNK, CHUNK)], bufM)
            pltpu.sync_copy(bufM, aggh_o.at[c, pl.ds(j * CHUNK, CHUNK)])

        return carry

    lax.fori_loop(0, (NSTAGE + NS - 1) // NS, fstage, 0)


@functools.cache
def _sc_scatter_h_kernel():
    return pl.kernel(
        _scatter_h_body,
        out_type=jax.ShapeDtypeStruct((NC, N, HID), jnp.float32),
        mesh=_sc_mesh(),
        scratch_types=[
            pltpu.VMEM((CHUNK,), jnp.int32),
            pltpu.VMEM((CHUNK, HID), jnp.float32),
            pltpu.VMEM_SHARED((N, HID), jnp.float32),
            pltpu.SemaphoreType.DMA,
        ],
    )


def _scatter_x_body(trans, rowi, aggx_o, idx, bufT, bufW, agg_x, sem):
    c = lax.axis_index("c")
    s = lax.axis_index("s")
    wid = s * NC + c
    base = wid * EPW

    def zrow(i, carry):
        for jc in range(HID // 16):
            bufW[i, pl.ds(jc * 16, 16)] = jnp.zeros((16,), jnp.float32)
        return carry

    lax.fori_loop(0, CHUNK, zrow, 0)

    def zstage(jj, carry):
        j = s + jj * NS

        @pl.when(j < NSTAGE)
        def _():
            pltpu.sync_copy(bufW, agg_x.at[pl.ds(j * CHUNK, CHUNK)])

        return carry

    lax.fori_loop(0, (NSTAGE + NS - 1) // NS, zstage, 0)
    plsc.subcore_barrier()

    def step(k, carry):
        off = base + k * CHUNK
        pltpu.sync_copy(rowi.at[pl.ds(off, CHUNK)], idx)
        pltpu.sync_copy(trans.at[pl.ds(off, CHUNK)], bufT)

        def widen(i, cc):
            bufW[i, pl.ds(0, XP)] = bufT[i, pl.ds(0, XP)]
            return cc

        lax.fori_loop(0, CHUNK, widen, 0)
        pltpu.sync_copy(bufW, agg_x.at[idx], add=True)
        return carry

    lax.fori_loop(0, NCHUNK, step, 0)
    plsc.subcore_barrier()

    def fstage(jj, carry):
        j = s + jj * NS

        @pl.when(j < NSTAGE)
        def _():
            pltpu.sync_copy(agg_x.at[pl.ds(j * CHUNK, CHUNK)], bufW)
            pltpu.sync_copy(bufW, aggx_o.at[c, pl.ds(j * CHUNK, CHUNK)])

        return carry

    lax.fori_loop(0, (NSTAGE + NS - 1) // NS, fstage, 0)


@functools.cache
def _sc_scatter_x_kernel():
    return pl.kernel(
        _scatter_x_body,
        out_type=jax.ShapeDtypeStruct((NC, N, HID), jnp.float32),
        mesh=_sc_mesh(),
        scratch_types=[
            pltpu.VMEM((CHUNK,), jnp.int32),
            pltpu.VMEM((CHUNK, XP), jnp.float32),
            pltpu.VMEM((CHUNK, HID), jnp.float32),
            pltpu.VMEM_SHARED((N, HID), jnp.float32),
            pltpu.SemaphoreType.DMA,
        ],
    )


def _edge_body(gA, gB, xd, ea, W0e, w0r, W1, b1, Wc0, bc0, wc1r,
               m_o, t_o):
    d = xd[...]
    radial = jnp.sum(d * d, axis=1, keepdims=True)
    feat = gA[...] + gB[...]
    feat += jnp.dot(ea[...], W0e[...], preferred_element_type=jnp.float32)
    feat += radial * w0r[...]
    m = jax.nn.silu(feat)
    m = jax.nn.silu(jnp.dot(m, W1[...], preferred_element_type=jnp.float32)
                    + b1[...])
    t = jax.nn.silu(jnp.dot(m, Wc0[...], preferred_element_type=jnp.float32)
                    + bc0[...])
    sc = jnp.sum(t * wc1r[...], axis=1, keepdims=True)
    trans = (d / jnp.sqrt(radial + 1e-8)) * sc
    m_o[...] = m
    t_o[...] = trans


def _full(shape):
    return pl.BlockSpec(shape, lambda i: (0, 0))


_tc_edge = pl.pallas_call(
    _edge_body,
    grid=(E // BT,),
    in_specs=[
        pl.BlockSpec((BT, HID), lambda i: (i, 0)),
        pl.BlockSpec((BT, HID), lambda i: (i, 0)),
        pl.BlockSpec((BT, XP), lambda i: (i, 0)),
        pl.BlockSpec((BT, DE), lambda i: (i, 0)),
        _full((DE, HID)),
        _full((1, HID)),
        _full((HID, HID)),
        _full((1, HID)),
        _full((HID, HID)),
        _full((1, HID)),
        _full((1, HID)),
    ],
    out_specs=[
        pl.BlockSpec((BT, HID), lambda i: (i, 0)),
        pl.BlockSpec((BT, XP), lambda i: (i, 0)),
    ],
    out_shape=[
        jax.ShapeDtypeStruct((E, HID), jnp.float32),
        jax.ShapeDtypeStruct((E, XP), jnp.float32),
    ],
    compiler_params=pltpu.CompilerParams(
        dimension_semantics=("parallel",)),
)


def _node_body(hh, ah0, ah1, x128, ax0, ax1,
               P1, P2, bn0, Wn1, bn1, WA, bA, WB,
               hh_o, x_o, hA_o, hB_o):
    aggh = ah0[0] + ah1[0]
    o = jax.nn.silu(
        jnp.dot(hh[...], P1[...], preferred_element_type=jnp.float32)
        + jnp.dot(aggh, P2[...], preferred_element_type=jnp.float32)
        + bn0[...])
    hn = hh[...] + jnp.dot(o, Wn1[...],
                           preferred_element_type=jnp.float32) + bn1[...]
    hh_o[...] = hn
    x_o[...] = x128[...] + ax0[0] + ax1[0]
    hA_o[...] = jnp.dot(hn, WA[...],
                        preferred_element_type=jnp.float32) + bA[...]
    hB_o[...] = jnp.dot(hn, WB[...], preferred_element_type=jnp.float32)


_tc_node = pl.pallas_call(
    _node_body,
    grid=(N // BN,),
    in_specs=[
        pl.BlockSpec((BN, HID), lambda i: (i, 0)),
        pl.BlockSpec((1, BN, HID), lambda i: (0, i, 0)),
        pl.BlockSpec((1, BN, HID), lambda i: (1, i, 0)),
        pl.BlockSpec((BN, HID), lambda i: (i, 0)),
        pl.BlockSpec((1, BN, HID), lambda i: (0, i, 0)),
        pl.BlockSpec((1, BN, HID), lambda i: (1, i, 0)),
        _full((HID, HID)),
        _full((HID, HID)),
        _full((1, HID)),
        _full((HID, HID)),
        _full((1, HID)),
        _full((HID, HID)),
        _full((1, HID)),
        _full((HID, HID)),
    ],
    out_specs=[
        pl.BlockSpec((BN, HID), lambda i: (i, 0)),
        pl.BlockSpec((BN, HID), lambda i: (i, 0)),
        pl.BlockSpec((BN, HID), lambda i: (i, 0)),
        pl.BlockSpec((BN, HID), lambda i: (i, 0)),
    ],
    out_shape=[
        jax.ShapeDtypeStruct((N, HID), jnp.float32),
        jax.ShapeDtypeStruct((N, HID), jnp.float32),
        jax.ShapeDtypeStruct((N, HID), jnp.float32),
        jax.ShapeDtypeStruct((N, HID), jnp.float32),
    ],
    compiler_params=pltpu.CompilerParams(
        dimension_semantics=("parallel",)),
)


def _embed_body(h, We, be, WA, bA, WB, hh_o, hA_o, hB_o):
    hh = jnp.dot(h[...], We[...], preferred_element_type=jnp.float32) + be[...]
    hh_o[...] = hh
    hA_o[...] = jnp.dot(hh, WA[...],
                        preferred_element_type=jnp.float32) + bA[...]
    hB_o[...] = jnp.dot(hh, WB[...], preferred_element_type=jnp.float32)


_tc_embed = pl.pallas_call(
    _embed_body,
    grid=(N // BN,),
    in_specs=[
        pl.BlockSpec((BN, D), lambda i: (i, 0)),
        _full((D, HID)),
        _full((1, HID)),
        _full((HID, HID)),
        _full((1, HID)),
        _full((HID, HID)),
    ],
    out_specs=[
        pl.BlockSpec((BN, HID), lambda i: (i, 0)),
        pl.BlockSpec((BN, HID), lambda i: (i, 0)),
        pl.BlockSpec((BN, HID), lambda i: (i, 0)),
    ],
    out_shape=[
        jax.ShapeDtypeStruct((N, HID), jnp.float32),
        jax.ShapeDtypeStruct((N, HID), jnp.float32),
        jax.ShapeDtypeStruct((N, HID), jnp.float32),
    ],
    compiler_params=pltpu.CompilerParams(
        dimension_semantics=("parallel",)),
)


def kernel(h, x, edges, edge_attr, params):
    row = edges[0]
    col = edges[1]
    x128 = jnp.pad(x, ((0, 0), (0, HID - 3)))
    layers = params["layers"]

    def w0_split(lp):
        W0 = lp["edge_mlp0"]["W"]
        b0 = lp["edge_mlp0"]["b"].reshape(1, HID)
        return (W0[:HID], b0, W0[HID:2 * HID], W0[2 * HID:2 * HID + 1],
                W0[2 * HID + 1:])

    WA0, bA0, WB0, _, _ = w0_split(layers[0])
    hh, hA, hB = _tc_embed(h, params["emb"]["W"],
                           params["emb"]["b"].reshape(1, HID), WA0, bA0, WB0)

    for i, lp in enumerate(layers):
        _, _, _, w0r, W0e = w0_split(lp)
        gA, gB, xd = _sc_gather_kernel()(hA, hB, x128, row, col)
        m, trans = _tc_edge(
            gA, gB, xd, edge_attr, W0e, w0r,
            lp["edge_mlp1"]["W"], lp["edge_mlp1"]["b"].reshape(1, HID),
            lp["coord_mlp0"]["W"], lp["coord_mlp0"]["b"].reshape(1, HID),
            lp["coord_mlp1"]["W"].reshape(1, HID))
        aggh = _sc_scatter_h_kernel()(m, row)
        aggx = _sc_scatter_x_kernel()(trans, row)
        if i + 1 < len(layers):
            WAn, bAn, WBn, _, _ = w0_split(layers[i + 1])
        else:
            WAn = params["emb_out"]["W"]
            bAn = params["emb_out"]["b"].reshape(1, D)
            WBn = jnp.zeros((HID, HID), jnp.float32)
        P = lp["node_mlp0"]["W"]
        hh, x128, hA, hB = _tc_node(
            hh, aggh, aggh, x128, aggx, aggx,
            P[:HID], P[HID:], lp["node_mlp0"]["b"].reshape(1, HID),
            lp["node_mlp1"]["W"], lp["node_mlp1"]["b"].reshape(1, HID),
            WAn, bAn, WBn)

    return (hA, x128[:, :3])


# speedup vs baseline: 2.1224x; 1.0412x over previous
import functools

import jax
import jax.numpy as jnp
from jax import lax
from jax.experimental import pallas as pl
from jax.experimental.pallas import tpu as pltpu
from jax.experimental.pallas import tpu_sc as plsc

N = 10000
E = 320000
D = 128
DE = 16
HID = 128
XP = 16

NC, NS = 2, 16
NW = NC * NS
EPW = E // NW
CHUNK = 80
NCHUNK = EPW // CHUNK
NSTAGE = N // CHUNK
GCH = 40
GNCHUNK = EPW // GCH
GNSTAGE = N // GCH

BT = 2000
BN = 2000


def _sc_mesh():
    return plsc.VectorSubcoreMesh(core_axis_name="c", subcore_axis_name="s",
                                  num_cores=NC, num_subcores=NS)


def _gather_body(hA, hB, x128, rowi, coli, gA_o, gB_o, xd_o,
                 idx_r, idx_c, bufA, bufB, bufxr, bufxc, bufd, x_sp, sem):
    c = lax.axis_index("c")
    s = lax.axis_index("s")
    wid = s * NC + c
    base = wid * EPW

    def stage(jj, carry):
        j = s + jj * NS

        @pl.when(j < GNSTAGE)
        def _():
            r0 = j * GCH
            pltpu.sync_copy(x128.at[pl.ds(r0, GCH)], bufxr)
            pltpu.sync_copy(bufxr, x_sp.at[pl.ds(r0, GCH)])

        return carry

    lax.fori_loop(0, (GNSTAGE + NS - 1) // NS, stage, 0)
    plsc.subcore_barrier()

    def step(k, carry):
        off = base + k * GCH
        pltpu.sync_copy(rowi.at[pl.ds(off, GCH)], idx_r)
        pltpu.sync_copy(coli.at[pl.ds(off, GCH)], idx_c)
        pltpu.async_copy(hA.at[idx_r], bufA, sem).wait()
        pltpu.async_copy(hB.at[idx_c], bufB, sem).wait()
        pltpu.async_copy(x_sp.at[idx_r], bufxr, sem).wait()
        pltpu.async_copy(x_sp.at[idx_c], bufxc, sem).wait()

        def diff(i, cc):
            bufd[i, pl.ds(0, XP)] = (bufxr[i, pl.ds(0, XP)]
                                     - bufxc[i, pl.ds(0, XP)])
            return cc

        lax.fori_loop(0, GCH, diff, 0)
        pltpu.sync_copy(bufA, gA_o.at[pl.ds(off, GCH)])
        pltpu.sync_copy(bufB, gB_o.at[pl.ds(off, GCH)])
        pltpu.sync_copy(bufd, xd_o.at[pl.ds(off, GCH)])
        return carry

    lax.fori_loop(0, GNCHUNK, step, 0)


@functools.cache
def _sc_gather_kernel():
    return pl.kernel(
        _gather_body,
        out_type=(
            jax.ShapeDtypeStruct((E, HID), jnp.float32),
            jax.ShapeDtypeStruct((E, HID), jnp.float32),
            jax.ShapeDtypeStruct((E, XP), jnp.float32),
        ),
        mesh=_sc_mesh(),
        scratch_types=[
            pltpu.VMEM((GCH,), jnp.int32),
            pltpu.VMEM((GCH,), jnp.int32),
            pltpu.VMEM((GCH, HID), jnp.float32),
            pltpu.VMEM((GCH, HID), jnp.float32),
            pltpu.VMEM((GCH, HID), jnp.float32),
            pltpu.VMEM((GCH, HID), jnp.float32),
            pltpu.VMEM((GCH, XP), jnp.float32),
            pltpu.VMEM_SHARED((N, HID), jnp.float32),
            pltpu.SemaphoreType.DMA,
        ],
    )


def _scatter_h_body(m, rowi, aggh_o, idx, bufM, agg_h, sem):
    c = lax.axis_index("c")
    s = lax.axis_index("s")
    wid = s * NC + c
    base = wid * EPW

    def zrow(i, carry):
        for jc in range(HID // 16):
            bufM[i, pl.ds(jc * 16, 16)] = jnp.zeros((16,), jnp.float32)
        return carry

    lax.fori_loop(0, CHUNK, zrow, 0)

    def zstage(jj, carry):
        j = s + jj * NS

        @pl.when(j < NSTAGE)
        def _():
            pltpu.sync_copy(bufM, agg_h.at[pl.ds(j * CHUNK, CHUNK)])

        return carry

    lax.fori_loop(0, (NSTAGE + NS - 1) // NS, zstage, 0)
    plsc.subcore_barrier()

    def step(k, carry):
        off = base + k * CHUNK
        pltpu.sync_copy(rowi.at[pl.ds(off, CHUNK)], idx)
        pltpu.sync_copy(m.at[pl.ds(off, CHUNK)], bufM)
        pltpu.sync_copy(bufM, agg_h.at[idx], add=True)
        return carry

    lax.fori_loop(0, NCHUNK, step, 0)
    plsc.subcore_barrier()

    def fstage(jj, carry):
        j = s + jj * NS

        @pl.when(j < NSTAGE)
        def _():
            pltpu.sync_copy(agg_h.at[pl.ds(j * CHUNK, CHUNK)], bufM)
            pltpu.sync_copy(bufM, aggh_o.at[c, pl.ds(j * CHUNK, CHUNK)])

        return carry

    lax.fori_loop(0, (NSTAGE + NS - 1) // NS, fstage, 0)


@functools.cache
def _sc_scatter_h_kernel():
    return pl.kernel(
        _scatter_h_body,
        out_type=jax.ShapeDtypeStruct((NC, N, HID), jnp.float32),
        mesh=_sc_mesh(),
        scratch_types=[
            pltpu.VMEM((CHUNK,), jnp.int32),
            pltpu.VMEM((CHUNK, HID), jnp.float32),
            pltpu.VMEM_SHARED((N, HID), jnp.float32),
            pltpu.SemaphoreType.DMA,
        ],
    )


def _scatter_x_body(trans, rowi, aggx_o, idx, bufT, agg_x, sem):
    c = lax.axis_index("c")
    s = lax.axis_index("s")
    wid = s * NC + c
    base = wid * EPW

    def zrow(i, carry):
        bufT[i, pl.ds(0, XP)] = jnp.zeros((XP,), jnp.float32)
        return carry

    lax.fori_loop(0, CHUNK, zrow, 0)

    def zstage(jj, carry):
        j = s + jj * NS

        @pl.when(j < NSTAGE)
        def _():
            pltpu.sync_copy(bufT, agg_x.at[pl.ds(j * CHUNK, CHUNK)])

        return carry

    lax.fori_loop(0, (NSTAGE + NS - 1) // NS, zstage, 0)
    plsc.subcore_barrier()

    def step(k, carry):
        off = base + k * CHUNK
        pltpu.sync_copy(rowi.at[pl.ds(off, CHUNK)], idx)
        pltpu.sync_copy(trans.at[pl.ds(off, CHUNK)], bufT)
        pltpu.sync_copy(bufT, agg_x.at[idx], add=True)
        return carry

    lax.fori_loop(0, NCHUNK, step, 0)
    plsc.subcore_barrier()

    def fstage(jj, carry):
        j = s + jj * NS

        @pl.when(j < NSTAGE)
        def _():
            pltpu.sync_copy(agg_x.at[pl.ds(j * CHUNK, CHUNK)], bufT)
            pltpu.sync_copy(bufT, aggx_o.at[c, pl.ds(j * CHUNK, CHUNK)])

        return carry

    lax.fori_loop(0, (NSTAGE + NS - 1) // NS, fstage, 0)


@functools.cache
def _sc_scatter_x_kernel():
    return pl.kernel(
        _scatter_x_body,
        out_type=jax.ShapeDtypeStruct((NC, N, XP), jnp.float32),
        mesh=_sc_mesh(),
        scratch_types=[
            pltpu.VMEM((CHUNK,), jnp.int32),
            pltpu.VMEM((CHUNK, XP), jnp.float32),
            pltpu.VMEM_SHARED((N, XP), jnp.float32),
            pltpu.SemaphoreType.DMA,
        ],
    )


def _edge_body(gA, gB, xd, ea, W0e, w0r, W1, b1, Wc0, bc0, wc1r,
               m_o, t_o):
    d = xd[...]
    radial = jnp.sum(d * d, axis=1, keepdims=True)
    feat = gA[...] + gB[...]
    feat += jnp.dot(ea[...], W0e[...], preferred_element_type=jnp.float32)
    feat += radial * w0r[...]
    m = jax.nn.silu(feat)
    m = jax.nn.silu(jnp.dot(m, W1[...], preferred_element_type=jnp.float32)
                    + b1[...])
    t = jax.nn.silu(jnp.dot(m, Wc0[...], preferred_element_type=jnp.float32)
                    + bc0[...])
    sc = jnp.sum(t * wc1r[...], axis=1, keepdims=True)
    trans = (d / jnp.sqrt(radial + 1e-8)) * sc
    m_o[...] = m
    t_o[...] = trans


def _full(shape):
    return pl.BlockSpec(shape, lambda i: (0, 0))


_tc_edge = pl.pallas_call(
    _edge_body,
    grid=(E // BT,),
    in_specs=[
        pl.BlockSpec((BT, HID), lambda i: (i, 0)),
        pl.BlockSpec((BT, HID), lambda i: (i, 0)),
        pl.BlockSpec((BT, XP), lambda i: (i, 0)),
        pl.BlockSpec((BT, DE), lambda i: (i, 0)),
        _full((DE, HID)),
        _full((1, HID)),
        _full((HID, HID)),
        _full((1, HID)),
        _full((HID, HID)),
        _full((1, HID)),
        _full((1, HID)),
    ],
    out_specs=[
        pl.BlockSpec((BT, HID), lambda i: (i, 0)),
        pl.BlockSpec((BT, XP), lambda i: (i, 0)),
    ],
    out_shape=[
        jax.ShapeDtypeStruct((E, HID), jnp.float32),
        jax.ShapeDtypeStruct((E, XP), jnp.float32),
    ],
    compiler_params=pltpu.CompilerParams(
        dimension_semantics=("parallel",)),
)


def _node_body(hh, ah0, ah1, x128, ax0, ax1,
               P1, P2, bn0, Wn1, bn1, WA, bA, WB,
               hh_o, x_o, hA_o, hB_o):
    aggh = ah0[0] + ah1[0]
    o = jax.nn.silu(
        jnp.dot(hh[...], P1[...], preferred_element_type=jnp.float32)
        + jnp.dot(aggh, P2[...], preferred_element_type=jnp.float32)
        + bn0[...])
    hn = hh[...] + jnp.dot(o, Wn1[...],
                           preferred_element_type=jnp.float32) + bn1[...]
    hh_o[...] = hn
    aggx = jnp.concatenate(
        [ax0[0] + ax1[0], jnp.zeros((BN, HID - XP), jnp.float32)], axis=1)
    x_o[...] = x128[...] + aggx
    hA_o[...] = jnp.dot(hn, WA[...],
                        preferred_element_type=jnp.float32) + bA[...]
    hB_o[...] = jnp.dot(hn, WB[...], preferred_element_type=jnp.float32)


_tc_node = pl.pallas_call(
    _node_body,
    grid=(N // BN,),
    in_specs=[
        pl.BlockSpec((BN, HID), lambda i: (i, 0)),
        pl.BlockSpec((1, BN, HID), lambda i: (0, i, 0)),
        pl.BlockSpec((1, BN, HID), lambda i: (1, i, 0)),
        pl.BlockSpec((BN, HID), lambda i: (i, 0)),
        pl.BlockSpec((1, BN, XP), lambda i: (0, i, 0)),
        pl.BlockSpec((1, BN, XP), lambda i: (1, i, 0)),
        _full((HID, HID)),
        _full((HID, HID)),
        _full((1, HID)),
        _full((HID, HID)),
        _full((1, HID)),
        _full((HID, HID)),
        _full((1, HID)),
        _full((HID, HID)),
    ],
    out_specs=[
        pl.BlockSpec((BN, HID), lambda i: (i, 0)),
        pl.BlockSpec((BN, HID), lambda i: (i, 0)),
        pl.BlockSpec((BN, HID), lambda i: (i, 0)),
        pl.BlockSpec((BN, HID), lambda i: (i, 0)),
    ],
    out_shape=[
        jax.ShapeDtypeStruct((N, HID), jnp.float32),
        jax.ShapeDtypeStruct((N, HID), jnp.float32),
        jax.ShapeDtypeStruct((N, HID), jnp.float32),
        jax.ShapeDtypeStruct((N, HID), jnp.float32),
    ],
    compiler_params=pltpu.CompilerParams(
        dimension_semantics=("parallel",)),
)


def _embed_body(h, We, be, WA, bA, WB, hh_o, hA_o, hB_o):
    hh = jnp.dot(h[...], We[...], preferred_element_type=jnp.float32) + be[...]
    hh_o[...] = hh
    hA_o[...] = jnp.dot(hh, WA[...],
                        preferred_element_type=jnp.float32) + bA[...]
    hB_o[...] = jnp.dot(hh, WB[...], preferred_element_type=jnp.float32)


_tc_embed = pl.pallas_call(
    _embed_body,
    grid=(N // BN,),
    in_specs=[
        pl.BlockSpec((BN, D), lambda i: (i, 0)),
        _full((D, HID)),
        _full((1, HID)),
        _full((HID, HID)),
        _full((1, HID)),
        _full((HID, HID)),
    ],
    out_specs=[
        pl.BlockSpec((BN, HID), lambda i: (i, 0)),
        pl.BlockSpec((BN, HID), lambda i: (i, 0)),
        pl.BlockSpec((BN, HID), lambda i: (i, 0)),
    ],
    out_shape=[
        jax.ShapeDtypeStruct((N, HID), jnp.float32),
        jax.ShapeDtypeStruct((N, HID), jnp.float32),
        jax.ShapeDtypeStruct((N, HID), jnp.float32),
    ],
    compiler_params=pltpu.CompilerParams(
        dimension_semantics=("parallel",)),
)


def kernel(h, x, edges, edge_attr, params):
    row = edges[0]
    col = edges[1]
    x128 = jnp.pad(x, ((0, 0), (0, HID - 3)))
    layers = params["layers"]

    def w0_split(lp):
        W0 = lp["edge_mlp0"]["W"]
        b0 = lp["edge_mlp0"]["b"].reshape(1, HID)
        return (W0[:HID], b0, W0[HID:2 * HID], W0[2 * HID:2 * HID + 1],
                W0[2 * HID + 1:])

    WA0, bA0, WB0, _, _ = w0_split(layers[0])
    hh, hA, hB = _tc_embed(h, params["emb"]["W"],
                           params["emb"]["b"].reshape(1, HID), WA0, bA0, WB0)

    for i, lp in enumerate(layers):
        _, _, _, w0r, W0e = w0_split(lp)
        gA, gB, xd = _sc_gather_kernel()(hA, hB, x128, row, col)
        m, trans = _tc_edge(
            gA, gB, xd, edge_attr, W0e, w0r,
            lp["edge_mlp1"]["W"], lp["edge_mlp1"]["b"].reshape(1, HID),
            lp["coord_mlp0"]["W"], lp["coord_mlp0"]["b"].reshape(1, HID),
            lp["coord_mlp1"]["W"].reshape(1, HID))
        aggh = _sc_scatter_h_kernel()(m, row)
        aggx = _sc_scatter_x_kernel()(trans, row)
        if i + 1 < len(layers):
            WAn, bAn, WBn, _, _ = w0_split(layers[i + 1])
        else:
            WAn = params["emb_out"]["W"]
            bAn = params["emb_out"]["b"].reshape(1, D)
            WBn = jnp.zeros((HID, HID), jnp.float32)
        P = lp["node_mlp0"]["W"]
        hh, x128, hA, hB = _tc_node(
            hh, aggh, aggh, x128, aggx, aggx,
            P[:HID], P[HID:], lp["node_mlp0"]["b"].reshape(1, HID),
            lp["node_mlp1"]["W"], lp["node_mlp1"]["b"].reshape(1, HID),
            WAn, bAn, WBn)

    return (hA, x128[:, :3])


# speedup vs baseline: 2.3295x; 1.0975x over previous
import functools

import jax
import jax.numpy as jnp
from jax import lax
from jax.experimental import pallas as pl
from jax.experimental.pallas import tpu as pltpu
from jax.experimental.pallas import tpu_sc as plsc

N = 10000
E = 320000
D = 128
DE = 16
HID = 128
XP = 16

NC, NS = 2, 16
NW = NC * NS
EPW = E // NW
CHUNK = 80
NCHUNK = EPW // CHUNK
NSTAGE = N // CHUNK
GCH = 40
GNCHUNK = EPW // GCH
GNSTAGE = N // GCH

BT = 2000
BN = 2000


def _sc_mesh():
    return plsc.VectorSubcoreMesh(core_axis_name="c", subcore_axis_name="s",
                                  num_cores=NC, num_subcores=NS)


def _gather_body(hA, hB, x128, rowi, coli, gA_o, gB_o, xd_o,
                 idx_r, idx_c, bufA, bufB, bufxr, bufxc, bufd, x_sp, sem):
    c = lax.axis_index("c")
    s = lax.axis_index("s")
    wid = s * NC + c
    base = wid * EPW

    def stage(jj, carry):
        j = s + jj * NS

        @pl.when(j < GNSTAGE)
        def _():
            r0 = j * GCH
            pltpu.sync_copy(x128.at[pl.ds(r0, GCH)], bufxr)
            pltpu.sync_copy(bufxr, x_sp.at[pl.ds(r0, GCH)])

        return carry

    lax.fori_loop(0, (GNSTAGE + NS - 1) // NS, stage, 0)
    plsc.subcore_barrier()

    def step(k, carry):
        off = base + k * GCH
        pltpu.sync_copy(rowi.at[pl.ds(off, GCH)], idx_r)
        pltpu.sync_copy(coli.at[pl.ds(off, GCH)], idx_c)
        pltpu.async_copy(hA.at[idx_r], bufA, sem).wait()
        pltpu.async_copy(hB.at[idx_c], bufB, sem).wait()
        pltpu.async_copy(x_sp.at[idx_r], bufxr, sem).wait()
        pltpu.async_copy(x_sp.at[idx_c], bufxc, sem).wait()

        def diff(i, cc):
            bufd[i, pl.ds(0, XP)] = (bufxr[i, pl.ds(0, XP)]
                                     - bufxc[i, pl.ds(0, XP)])
            return cc

        lax.fori_loop(0, GCH, diff, 0)
        pltpu.sync_copy(bufA, gA_o.at[pl.ds(off, GCH)])
        pltpu.sync_copy(bufB, gB_o.at[pl.ds(off, GCH)])
        pltpu.sync_copy(bufd, xd_o.at[pl.ds(off, GCH)])
        return carry

    lax.fori_loop(0, GNCHUNK, step, 0)


@functools.cache
def _sc_gather_kernel():
    return pl.kernel(
        _gather_body,
        out_type=(
            jax.ShapeDtypeStruct((E, HID), jnp.float32),
            jax.ShapeDtypeStruct((E, HID), jnp.float32),
            jax.ShapeDtypeStruct((E, XP), jnp.float32),
        ),
        mesh=_sc_mesh(),
        scratch_types=[
            pltpu.VMEM((GCH,), jnp.int32),
            pltpu.VMEM((GCH,), jnp.int32),
            pltpu.VMEM((GCH, HID), jnp.float32),
            pltpu.VMEM((GCH, HID), jnp.float32),
            pltpu.VMEM((GCH, HID), jnp.float32),
            pltpu.VMEM((GCH, HID), jnp.float32),
            pltpu.VMEM((GCH, XP), jnp.float32),
            pltpu.VMEM_SHARED((N, HID), jnp.float32),
            pltpu.SemaphoreType.DMA,
        ],
    )


def _scatter_h_body(m, rowi, aggh_o, idx, bufM, agg_h, sem):
    c = lax.axis_index("c")
    s = lax.axis_index("s")
    wid = s * NC + c
    base = wid * EPW

    def zrow(i, carry):
        for jc in range(HID // 16):
            bufM[0, i, pl.ds(jc * 16, 16)] = jnp.zeros((16,), jnp.float32)
        return carry

    lax.fori_loop(0, CHUNK, zrow, 0)

    def zstage(jj, carry):
        j = s + jj * NS

        @pl.when(j < NSTAGE)
        def _():
            pltpu.sync_copy(bufM.at[0], agg_h.at[pl.ds(j * CHUNK, CHUNK)])

        return carry

    lax.fori_loop(0, (NSTAGE + NS - 1) // NS, zstage, 0)
    plsc.subcore_barrier()

    pltpu.sync_copy(rowi.at[pl.ds(base, CHUNK)], idx.at[0])
    pltpu.sync_copy(m.at[pl.ds(base, CHUNK)], bufM.at[0])

    def step(k, carry):
        p = lax.rem(k, 2)
        q = lax.rem(k + 1, 2)

        @pl.when(k + 1 < NCHUNK)
        def _():
            off2 = base + (k + 1) * CHUNK
            h1 = pltpu.async_copy(rowi.at[pl.ds(off2, CHUNK)], idx.at[q], sem)
            h2 = pltpu.async_copy(m.at[pl.ds(off2, CHUNK)], bufM.at[q], sem)
            pltpu.sync_copy(bufM.at[p], agg_h.at[idx.at[p]], add=True)
            h1.wait()
            h2.wait()

        @pl.when(k + 1 >= NCHUNK)
        def _():
            pltpu.sync_copy(bufM.at[p], agg_h.at[idx.at[p]], add=True)

        return carry

    lax.fori_loop(0, NCHUNK, step, 0)
    plsc.subcore_barrier()

    def fstage(jj, carry):
        j = s + jj * NS

        @pl.when(j < NSTAGE)
        def _():
            pltpu.sync_copy(agg_h.at[pl.ds(j * CHUNK, CHUNK)], bufM.at[0])
            pltpu.sync_copy(bufM.at[0], aggh_o.at[c, pl.ds(j * CHUNK, CHUNK)])

        return carry

    lax.fori_loop(0, (NSTAGE + NS - 1) // NS, fstage, 0)


@functools.cache
def _sc_scatter_h_kernel():
    return pl.kernel(
        _scatter_h_body,
        out_type=jax.ShapeDtypeStruct((NC, N, HID), jnp.float32),
        mesh=_sc_mesh(),
        scratch_types=[
            pltpu.VMEM((2, CHUNK), jnp.int32),
            pltpu.VMEM((2, CHUNK, HID), jnp.float32),
            pltpu.VMEM_SHARED((N, HID), jnp.float32),
            pltpu.SemaphoreType.DMA,
        ],
    )


def _scatter_x_body(trans, rowi, aggx_o, idx, bufT, agg_x, sem):
    c = lax.axis_index("c")
    s = lax.axis_index("s")
    wid = s * NC + c
    base = wid * EPW

    def zrow(i, carry):
        bufT[0, i, pl.ds(0, XP)] = jnp.zeros((XP,), jnp.float32)
        return carry

    lax.fori_loop(0, CHUNK, zrow, 0)

    def zstage(jj, carry):
        j = s + jj * NS

        @pl.when(j < NSTAGE)
        def _():
            pltpu.sync_copy(bufT.at[0], agg_x.at[pl.ds(j * CHUNK, CHUNK)])

        return carry

    lax.fori_loop(0, (NSTAGE + NS - 1) // NS, zstage, 0)
    plsc.subcore_barrier()

    pltpu.sync_copy(rowi.at[pl.ds(base, CHUNK)], idx.at[0])
    pltpu.sync_copy(trans.at[pl.ds(base, CHUNK)], bufT.at[0])

    def step(k, carry):
        p = lax.rem(k, 2)
        q = lax.rem(k + 1, 2)

        @pl.when(k + 1 < NCHUNK)
        def _():
            off2 = base + (k + 1) * CHUNK
            h1 = pltpu.async_copy(rowi.at[pl.ds(off2, CHUNK)], idx.at[q], sem)
            h2 = pltpu.async_copy(trans.at[pl.ds(off2, CHUNK)], bufT.at[q],
                                  sem)
            pltpu.sync_copy(bufT.at[p], agg_x.at[idx.at[p]], add=True)
            h1.wait()
            h2.wait()

        @pl.when(k + 1 >= NCHUNK)
        def _():
            pltpu.sync_copy(bufT.at[p], agg_x.at[idx.at[p]], add=True)

        return carry

    lax.fori_loop(0, NCHUNK, step, 0)
    plsc.subcore_barrier()

    def fstage(jj, carry):
        j = s + jj * NS

        @pl.when(j < NSTAGE)
        def _():
            pltpu.sync_copy(agg_x.at[pl.ds(j * CHUNK, CHUNK)], bufT.at[0])
            pltpu.sync_copy(bufT.at[0], aggx_o.at[c, pl.ds(j * CHUNK, CHUNK)])

        return carry

    lax.fori_loop(0, (NSTAGE + NS - 1) // NS, fstage, 0)


@functools.cache
def _sc_scatter_x_kernel():
    return pl.kernel(
        _scatter_x_body,
        out_type=jax.ShapeDtypeStruct((NC, N, XP), jnp.float32),
        mesh=_sc_mesh(),
        scratch_types=[
            pltpu.VMEM((2, CHUNK), jnp.int32),
            pltpu.VMEM((2, CHUNK, XP), jnp.float32),
            pltpu.VMEM_SHARED((N, XP), jnp.float32),
            pltpu.SemaphoreType.DMA,
        ],
    )


def _edge_body(gA, gB, xd, ea, W0e, w0r, W1, b1, Wc0, bc0, wc1r,
               m_o, t_o):
    d = xd[...]
    radial = jnp.sum(d * d, axis=1, keepdims=True)
    feat = gA[...] + gB[...]
    feat += jnp.dot(ea[...], W0e[...], preferred_element_type=jnp.float32)
    feat += radial * w0r[...]
    m = jax.nn.silu(feat)
    m = jax.nn.silu(jnp.dot(m, W1[...], preferred_element_type=jnp.float32)
                    + b1[...])
    t = jax.nn.silu(jnp.dot(m, Wc0[...], preferred_element_type=jnp.float32)
                    + bc0[...])
    sc = jnp.sum(t * wc1r[...], axis=1, keepdims=True)
    trans = (d / jnp.sqrt(radial + 1e-8)) * sc
    m_o[...] = m
    t_o[...] = trans


def _full(shape):
    return pl.BlockSpec(shape, lambda i: (0, 0))


_tc_edge = pl.pallas_call(
    _edge_body,
    grid=(E // BT,),
    in_specs=[
        pl.BlockSpec((BT, HID), lambda i: (i, 0)),
        pl.BlockSpec((BT, HID), lambda i: (i, 0)),
        pl.BlockSpec((BT, XP), lambda i: (i, 0)),
        pl.BlockSpec((BT, DE), lambda i: (i, 0)),
        _full((DE, HID)),
        _full((1, HID)),
        _full((HID, HID)),
        _full((1, HID)),
        _full((HID, HID)),
        _full((1, HID)),
        _full((1, HID)),
    ],
    out_specs=[
        pl.BlockSpec((BT, HID), lambda i: (i, 0)),
        pl.BlockSpec((BT, XP), lambda i: (i, 0)),
    ],
    out_shape=[
        jax.ShapeDtypeStruct((E, HID), jnp.float32),
        jax.ShapeDtypeStruct((E, XP), jnp.float32),
    ],
    compiler_params=pltpu.CompilerParams(
        dimension_semantics=("parallel",)),
)


def _node_body(hh, ah0, ah1, x128, ax0, ax1,
               P1, P2, bn0, Wn1, bn1, WA, bA, WB,
               hh_o, x_o, hA_o, hB_o):
    aggh = ah0[0] + ah1[0]
    o = jax.nn.silu(
        jnp.dot(hh[...], P1[...], preferred_element_type=jnp.float32)
        + jnp.dot(aggh, P2[...], preferred_element_type=jnp.float32)
        + bn0[...])
    hn = hh[...] + jnp.dot(o, Wn1[...],
                           preferred_element_type=jnp.float32) + bn1[...]
    hh_o[...] = hn
    aggx = jnp.concatenate(
        [ax0[0] + ax1[0], jnp.zeros((BN, HID - XP), jnp.float32)], axis=1)
    x_o[...] = x128[...] + aggx
    hA_o[...] = jnp.dot(hn, WA[...],
                        preferred_element_type=jnp.float32) + bA[...]
    hB_o[...] = jnp.dot(hn, WB[...], preferred_element_type=jnp.float32)


_tc_node = pl.pallas_call(
    _node_body,
    grid=(N // BN,),
    in_specs=[
        pl.BlockSpec((BN, HID), lambda i: (i, 0)),
        pl.BlockSpec((1, BN, HID), lambda i: (0, i, 0)),
        pl.BlockSpec((1, BN, HID), lambda i: (1, i, 0)),
        pl.BlockSpec((BN, HID), lambda i: (i, 0)),
        pl.BlockSpec((1, BN, XP), lambda i: (0, i, 0)),
        pl.BlockSpec((1, BN, XP), lambda i: (1, i, 0)),
        _full((HID, HID)),
        _full((HID, HID)),
        _full((1, HID)),
        _full((HID, HID)),
        _full((1, HID)),
        _full((HID, HID)),
        _full((1, HID)),
        _full((HID, HID)),
    ],
    out_specs=[
        pl.BlockSpec((BN, HID), lambda i: (i, 0)),
        pl.BlockSpec((BN, HID), lambda i: (i, 0)),
        pl.BlockSpec((BN, HID), lambda i: (i, 0)),
        pl.BlockSpec((BN, HID), lambda i: (i, 0)),
    ],
    out_shape=[
        jax.ShapeDtypeStruct((N, HID), jnp.float32),
        jax.ShapeDtypeStruct((N, HID), jnp.float32),
        jax.ShapeDtypeStruct((N, HID), jnp.float32),
        jax.ShapeDtypeStruct((N, HID), jnp.float32),
    ],
    compiler_params=pltpu.CompilerParams(
        dimension_semantics=("parallel",)),
)


def _embed_body(h, We, be, WA, bA, WB, hh_o, hA_o, hB_o):
    hh = jnp.dot(h[...], We[...], preferred_element_type=jnp.float32) + be[...]
    hh_o[...] = hh
    hA_o[...] = jnp.dot(hh, WA[...],
                        preferred_element_type=jnp.float32) + bA[...]
    hB_o[...] = jnp.dot(hh, WB[...], preferred_element_type=jnp.float32)


_tc_embed = pl.pallas_call(
    _embed_body,
    grid=(N // BN,),
    in_specs=[
        pl.BlockSpec((BN, D), lambda i: (i, 0)),
        _full((D, HID)),
        _full((1, HID)),
        _full((HID, HID)),
        _full((1, HID)),
        _full((HID, HID)),
    ],
    out_specs=[
        pl.BlockSpec((BN, HID), lambda i: (i, 0)),
        pl.BlockSpec((BN, HID), lambda i: (i, 0)),
        pl.BlockSpec((BN, HID), lambda i: (i, 0)),
    ],
    out_shape=[
        jax.ShapeDtypeStruct((N, HID), jnp.float32),
        jax.ShapeDtypeStruct((N, HID), jnp.float32),
        jax.ShapeDtypeStruct((N, HID), jnp.float32),
    ],
    compiler_params=pltpu.CompilerParams(
        dimension_semantics=("parallel",)),
)


def kernel(h, x, edges, edge_attr, params):
    row = edges[0]
    col = edges[1]
    x128 = jnp.pad(x, ((0, 0), (0, HID - 3)))
    layers = params["layers"]

    def w0_split(lp):
        W0 = lp["edge_mlp0"]["W"]
        b0 = lp["edge_mlp0"]["b"].reshape(1, HID)
        return (W0[:HID], b0, W0[HID:2 * HID], W0[2 * HID:2 * HID + 1],
                W0[2 * HID + 1:])

    WA0, bA0, WB0, _, _ = w0_split(layers[0])
    hh, hA, hB = _tc_embed(h, params["emb"]["W"],
                           params["emb"]["b"].reshape(1, HID), WA0, bA0, WB0)

    for i, lp in enumerate(layers):
        _, _, _, w0r, W0e = w0_split(lp)
        gA, gB, xd = _sc_gather_kernel()(hA, hB, x128, row, col)
        m, trans = _tc_edge(
            gA, gB, xd, edge_attr, W0e, w0r,
            lp["edge_mlp1"]["W"], lp["edge_mlp1"]["b"].reshape(1, HID),
            lp["coord_mlp0"]["W"], lp["coord_mlp0"]["b"].reshape(1, HID),
            lp["coord_mlp1"]["W"].reshape(1, HID))
        aggh = _sc_scatter_h_kernel()(m, row)
        aggx = _sc_scatter_x_kernel()(trans, row)
        if i + 1 < len(layers):
            WAn, bAn, WBn, _, _ = w0_split(layers[i + 1])
        else:
            WAn = params["emb_out"]["W"]
            bAn = params["emb_out"]["b"].reshape(1, D)
            WBn = jnp.zeros((HID, HID), jnp.float32)
        P = lp["node_mlp0"]["W"]
        hh, x128, hA, hB = _tc_node(
            hh, aggh, aggh, x128, aggx, aggx,
            P[:HID], P[HID:], lp["node_mlp0"]["b"].reshape(1, HID),
            lp["node_mlp1"]["W"], lp["node_mlp1"]["b"].reshape(1, HID),
            WAn, bAn, WBn)

    return (hA, x128[:, :3])
